# Initial kernel scaffold; baseline (speedup 1.0000x reference)
import jax
import jax.numpy as jnp
from jax.experimental import pallas as pl


def kernel(substructure_indices, embedding_table):
    raise NotImplementedError("write your pallas kernel here")


# speedup vs baseline: 15.8868x; 15.8868x over previous
import jax
import jax.numpy as jnp
from jax import lax
from jax.experimental import pallas as pl
from jax.experimental.pallas import tpu as pltpu
from jax.experimental.pallas import tpu_sc as plsc

NC = 2
NS = 16
NW = NC * NS

VOCAB = 194
DIM = 128
B_TOTAL = 4096 * 200
B_PER_W = B_TOTAL // NW
CHUNK = 128
N_CHUNKS = B_PER_W // CHUNK
NBUF = 4
LOOKAHEAD = NBUF // 2


def _body(table_hbm, idx_hbm, out_hbm,
          idx_v, rows0, rows1, rows2, rows3, table_sh,
          g0, g1, g2, g3, o0, o1, o2, o3):
    rows = (rows0, rows1, rows2, rows3)
    gsem = (g0, g1, g2, g3)
    osem = (o0, o1, o2, o3)

    cid = lax.axis_index("c")
    sid = lax.axis_index("s")
    wid = sid * NC + cid
    out_base = wid * B_PER_W

    @pl.when(sid == 0)
    def _():
        pltpu.sync_copy(table_hbm, table_sh)

    plsc.subcore_barrier()

    pltpu.sync_copy(idx_hbm.at[pl.ds(wid * N_CHUNKS, N_CHUNKS)], idx_v)

    def start_gather(j, b):
        pltpu.make_async_copy(table_sh.at[idx_v.at[j]], rows[b], gsem[b]).start()

    def wait_gather(j, b):
        pltpu.make_async_copy(table_sh.at[idx_v.at[j]], rows[b], gsem[b]).wait()

    def start_out(j, b):
        pltpu.make_async_copy(
            rows[b], out_hbm.at[pl.ds(out_base + j * CHUNK, CHUNK)], osem[b]
        ).start()

    def wait_out(j, b):
        pltpu.make_async_copy(
            rows[b], out_hbm.at[pl.ds(out_base + j * CHUNK, CHUNK)], osem[b]
        ).wait()

    for b in range(LOOKAHEAD):
        start_gather(b, b)
    for j in range(LOOKAHEAD):
        wait_gather(j, j)
        start_out(j, j)
        start_gather(j + LOOKAHEAD, j + LOOKAHEAD)

    n_steady = N_CHUNKS - 2 * LOOKAHEAD
    n_groups = n_steady // NBUF

    def group(g, _):
        j0 = LOOKAHEAD + g * NBUF
        for b in range(NBUF):
            j = j0 + b
            wb = (b + LOOKAHEAD) % NBUF
            wait_gather(j, wb)
            start_out(j, wb)
            wait_out(j - LOOKAHEAD, b)
            start_gather(j + LOOKAHEAD, b)
        return 0

    lax.fori_loop(0, n_groups, group, 0)

    for j in range(N_CHUNKS - LOOKAHEAD, N_CHUNKS):
        b = j % NBUF
        wait_gather(j, b)
        start_out(j, b)
        wait_out(j - LOOKAHEAD, (j - LOOKAHEAD) % NBUF)
    for j in range(N_CHUNKS - LOOKAHEAD, N_CHUNKS):
        wait_out(j, j % NBUF)


def kernel(substructure_indices, embedding_table):
    idx2d = substructure_indices.astype(jnp.int32).reshape(NW * N_CHUNKS, CHUNK)
    mesh = plsc.VectorSubcoreMesh(core_axis_name="c", subcore_axis_name="s")
    out = pl.kernel(
        _body,
        out_type=jax.ShapeDtypeStruct((B_TOTAL, DIM), jnp.float32),
        mesh=mesh,
        scratch_types=[
            pltpu.VMEM((N_CHUNKS, CHUNK), jnp.int32),
            pltpu.VMEM((CHUNK, DIM), jnp.float32),
            pltpu.VMEM((CHUNK, DIM), jnp.float32),
            pltpu.VMEM((CHUNK, DIM), jnp.float32),
            pltpu.VMEM((CHUNK, DIM), jnp.float32),
            pltpu.VMEM_SHARED((VOCAB, DIM), jnp.float32),
            pltpu.SemaphoreType.DMA,
            pltpu.SemaphoreType.DMA,
            pltpu.SemaphoreType.DMA,
            pltpu.SemaphoreType.DMA,
            pltpu.SemaphoreType.DMA,
            pltpu.SemaphoreType.DMA,
            pltpu.SemaphoreType.DMA,
            pltpu.SemaphoreType.DMA,
        ],
    )(embedding_table, idx2d)
    return out.reshape(4096, 200, DIM)


# speedup vs baseline: 15.8951x; 1.0005x over previous
import jax
import jax.numpy as jnp
from jax import lax
from jax.experimental import pallas as pl
from jax.experimental.pallas import tpu as pltpu
from jax.experimental.pallas import tpu_sc as plsc

NC = 2
NS = 16
NW = NC * NS

VOCAB = 194
DIM = 128
B_TOTAL = 4096 * 200
B_PER_W = B_TOTAL // NW
CHUNK = 128
N_CHUNKS = B_PER_W // CHUNK
NBUF = 5
LOOKAHEAD = 2

_STEADY_LO = NBUF - LOOKAHEAD
_STEADY_HI = N_CHUNKS - LOOKAHEAD
assert (_STEADY_HI - _STEADY_LO) % NBUF == 0
_N_GROUPS = (_STEADY_HI - _STEADY_LO) // NBUF


def _body(table_hbm, idx_hbm, out_hbm, idx_v, *rest):
    rows = rest[:NBUF]
    table_sh = rest[NBUF]
    gsem = rest[NBUF + 1:2 * NBUF + 1]
    osem = rest[2 * NBUF + 1:]

    cid = lax.axis_index("c")
    sid = lax.axis_index("s")
    wid = sid * NC + cid
    out_base = wid * B_PER_W

    @pl.when(sid == 0)
    def _():
        pltpu.sync_copy(table_hbm, table_sh)

    plsc.subcore_barrier()

    pltpu.sync_copy(idx_hbm.at[pl.ds(wid * N_CHUNKS, N_CHUNKS)], idx_v)

    def start_gather(j, b):
        pltpu.make_async_copy(table_sh.at[idx_v.at[j]], rows[b], gsem[b]).start()

    def wait_gather(j, b):
        pltpu.make_async_copy(table_sh.at[idx_v.at[j]], rows[b], gsem[b]).wait()

    def start_out(j, b):
        pltpu.make_async_copy(
            rows[b], out_hbm.at[pl.ds(out_base + j * CHUNK, CHUNK)], osem[b]
        ).start()

    def wait_out(j, b):
        pltpu.make_async_copy(
            rows[b], out_hbm.at[pl.ds(out_base + j * CHUNK, CHUNK)], osem[b]
        ).wait()

    for b in range(LOOKAHEAD):
        start_gather(b, b)
    for j in range(_STEADY_LO):
        wait_gather(j, j % NBUF)
        start_out(j, j % NBUF)
        start_gather(j + LOOKAHEAD, (j + LOOKAHEAD) % NBUF)

    def group(g, _):
        j0 = _STEADY_LO + g * NBUF
        for b in range(NBUF):
            j = j0 + b
            wb = (_STEADY_LO + b) % NBUF
            pb = (_STEADY_LO + b + LOOKAHEAD) % NBUF
            wait_gather(j, wb)
            start_out(j, wb)
            wait_out(j - (NBUF - LOOKAHEAD), pb)
            start_gather(j + LOOKAHEAD, pb)
        return 0

    lax.fori_loop(0, _N_GROUPS, group, 0)

    for j in range(_STEADY_HI, N_CHUNKS):
        wait_gather(j, j % NBUF)
        start_out(j, j % NBUF)
    for j in range(N_CHUNKS - NBUF, N_CHUNKS):
        wait_out(j, j % NBUF)


def kernel(substructure_indices, embedding_table):
    idx2d = substructure_indices.astype(jnp.int32).reshape(NW * N_CHUNKS, CHUNK)
    mesh = plsc.VectorSubcoreMesh(core_axis_name="c", subcore_axis_name="s")
    out = pl.kernel(
        _body,
        out_type=jax.ShapeDtypeStruct((B_TOTAL, DIM), jnp.float32),
        mesh=mesh,
        scratch_types=[
            pltpu.VMEM((N_CHUNKS, CHUNK), jnp.int32),
            *[pltpu.VMEM((CHUNK, DIM), jnp.float32) for _ in range(NBUF)],
            pltpu.VMEM_SHARED((VOCAB, DIM), jnp.float32),
            *[pltpu.SemaphoreType.DMA for _ in range(2 * NBUF)],
        ],
    )(embedding_table, idx2d)
    return out.reshape(4096, 200, DIM)


# speedup vs baseline: 15.9796x; 1.0053x over previous
import jax
import jax.numpy as jnp
from jax import lax
from jax.experimental import pallas as pl
from jax.experimental.pallas import tpu as pltpu
from jax.experimental.pallas import tpu_sc as plsc

NC = 2
NS = 16
NW = NC * NS

VOCAB = 194
DIM = 128
B_TOTAL = 4096 * 200
B_PER_W = B_TOTAL // NW
CHUNK = 128
N_CHUNKS = B_PER_W // CHUNK
NBUF = 6
G = 3

_STEADY_LO = NBUF - G
_N_GROUPS = (N_CHUNKS - NBUF) // NBUF
_STEADY_HI = _STEADY_LO + _N_GROUPS * NBUF


def _body(table_hbm, idx_hbm, out_hbm, idx_v, *rest):
    rows = rest[:NBUF]
    table_sh = rest[NBUF]
    gsem = rest[NBUF + 1:2 * NBUF + 1]
    osem = rest[2 * NBUF + 1:]

    cid = lax.axis_index("c")
    sid = lax.axis_index("s")
    wid = sid * NC + cid
    out_base = wid * B_PER_W

    @pl.when(sid == 0)
    def _():
        pltpu.sync_copy(table_hbm, table_sh)

    plsc.subcore_barrier()

    pltpu.sync_copy(idx_hbm.at[pl.ds(wid * N_CHUNKS, N_CHUNKS)], idx_v)

    def start_gather(j, b):
        pltpu.make_async_copy(table_sh.at[idx_v.at[j]], rows[b], gsem[b]).start()

    def wait_gather(j, b):
        pltpu.make_async_copy(table_sh.at[idx_v.at[j]], rows[b], gsem[b]).wait()

    def start_out(j, b):
        pltpu.make_async_copy(
            rows[b], out_hbm.at[pl.ds(out_base + j * CHUNK, CHUNK)], osem[b]
        ).start()

    def wait_out(j, b):
        pltpu.make_async_copy(
            rows[b], out_hbm.at[pl.ds(out_base + j * CHUNK, CHUNK)], osem[b]
        ).wait()

    for b in range(G):
        start_gather(b, b)
    for j in range(_STEADY_LO):
        wait_gather(j, j)
        start_out(j, j)
        start_gather(j + G, (j + G) % NBUF)

    def group(g, _):
        j0 = _STEADY_LO + g * NBUF
        for b in range(NBUF):
            j = j0 + b
            wb = (_STEADY_LO + b) % NBUF
            wait_out(j - G, b)
            start_gather(j + G, b)
            wait_gather(j, wb)
            start_out(j, wb)
        return 0

    lax.fori_loop(0, _N_GROUPS, group, 0)

    for j in range(_STEADY_HI, N_CHUNKS):
        if j + G < N_CHUNKS:
            wait_out(j - G, (j + G) % NBUF)
            start_gather(j + G, (j + G) % NBUF)
        wait_gather(j, j % NBUF)
        start_out(j, j % NBUF)
    for j in range(N_CHUNKS - NBUF, N_CHUNKS):
        wait_out(j, j % NBUF)


def kernel(substructure_indices, embedding_table):
    idx2d = substructure_indices.astype(jnp.int32).reshape(NW * N_CHUNKS, CHUNK)
    mesh = plsc.VectorSubcoreMesh(core_axis_name="c", subcore_axis_name="s")
    out = pl.kernel(
        _body,
        out_type=jax.ShapeDtypeStruct((B_TOTAL, DIM), jnp.float32),
        mesh=mesh,
        scratch_types=[
            pltpu.VMEM((N_CHUNKS, CHUNK), jnp.int32),
            *[pltpu.VMEM((CHUNK, DIM), jnp.float32) for _ in range(NBUF)],
            pltpu.VMEM_SHARED((VOCAB, DIM), jnp.float32),
            *[pltpu.SemaphoreType.DMA for _ in range(2 * NBUF)],
        ],
    )(embedding_table, idx2d)
    return out.reshape(4096, 200, DIM)
